# baseline (speedup 1.0000x reference)
import jax
import jax.numpy as jnp
from jax import lax
from jax.experimental import pallas as pl
from jax.experimental.pallas import tpu as pltpu
from jax.experimental.pallas import tpu_sc as plsc

_N, _K, _H, _W = 1024, 17, 64, 64
_NE = _N * _K
_NWORK = 32
_PER_W = _NE // _NWORK
_VREGS = _PER_W // 16
_IDX_ROWS = 5
_L = 16
_MAGIC = 8388608.0


def _step01(v):
    return jnp.minimum(jnp.sign(v) + 1.0, 1.0)


def _body(pred_hbm, score_hbm, t_hbm, out_hbm,
          pred_v, score_v, idx_v, gath_v, mf_v, acc_v, sem):
    c = lax.axis_index("c")
    s = lax.axis_index("s")
    wid = c * 16 + s
    ebase = wid * _PER_W

    pltpu.sync_copy(pred_hbm.at[pl.ds(ebase * 2, _PER_W * 2)], pred_v)
    pltpu.sync_copy(score_hbm.at[pl.ds(ebase, _PER_W)], score_v)

    lane = lax.iota(jnp.int32, 16)
    for j in range(_VREGS):
        e = j * _L + lane
        x = plsc.load_gather(pred_v, [e * 2])
        y = plsc.load_gather(pred_v, [e * 2 + 1])
        vx = jnp.minimum(jnp.maximum(x, 0.0), 1.0) * (_W - 1)
        vy = jnp.minimum(jnp.maximum(y, 0.0), 1.0) * (_H - 1)
        ix = ((vx + _MAGIC) - _MAGIC).astype(jnp.int32)
        iy = ((vy + _MAGIC) - _MAGIC).astype(jnp.int32)
        mf = (_step01(x) * _step01(1.0 - x) * _step01(y) * _step01(1.0 - y))
        idx_v[j // 8, pl.ds((j % 8) * _L, _L)] = (
            (ebase + e) * (_H * _W) + iy * _W + ix)
        mf_v[pl.ds(j * _L, _L)] = mf
    for j in range(_VREGS, _IDX_ROWS * 8):
        idx_v[j // 8, pl.ds((j % 8) * _L, _L)] = jnp.zeros((_L,), jnp.int32)

    copies = [pltpu.async_copy(t_hbm.at[idx_v.at[r]], gath_v.at[r], sem)
              for r in range(_IDX_ROWS)]
    for cp in copies:
        cp.wait()

    acc = jnp.zeros((_L,), jnp.float32)
    for j in range(_VREGS):
        g = gath_v[j // 8, pl.ds((j % 8) * _L, _L)]
        mf = mf_v[pl.ds(j * _L, _L)]
        sv = score_v[pl.ds(j * _L, _L)]
        acc = acc + jnp.abs(sv - g * mf)
    acc_v[...] = acc
    pltpu.sync_copy(acc_v, out_hbm.at[pl.ds(wid * _L, _L)])


_mesh = plsc.VectorSubcoreMesh(core_axis_name="c", subcore_axis_name="s")

_sc_call = pl.kernel(
    _body,
    out_type=jax.ShapeDtypeStruct((_NWORK * _L,), jnp.float32),
    mesh=_mesh,
    compiler_params=pltpu.CompilerParams(needs_layout_passes=False),
    scratch_types=[
        pltpu.VMEM((_PER_W * 2,), jnp.float32),
        pltpu.VMEM((_PER_W,), jnp.float32),
        pltpu.VMEM((_IDX_ROWS, 128), jnp.int32),
        pltpu.VMEM((_IDX_ROWS, 128), jnp.float32),
        pltpu.VMEM((_PER_W,), jnp.float32),
        pltpu.VMEM((_L,), jnp.float32),
        pltpu.SemaphoreType.DMA,
    ],
)


def kernel(pred, score, teacher_output):
    pred_flat = pred.reshape(-1)
    score_flat = score.reshape(-1)
    t_flat = teacher_output.reshape(-1)
    partials = _sc_call(pred_flat, score_flat, t_flat)
    return jnp.sum(partials) * (1.0 / _NE)
